# Initial kernel scaffold; baseline (speedup 1.0000x reference)
import jax
import jax.numpy as jnp
from jax.experimental import pallas as pl


def kernel(coords, bias):
    raise NotImplementedError("write your pallas kernel here")


# speedup vs baseline: 31.1596x; 31.1596x over previous
import functools

import numpy as np
import jax
import jax.numpy as jnp
from jax import lax
from jax.experimental import pallas as pl
from jax.experimental.pallas import tpu as pltpu
from jax.experimental.pallas import tpu_sc as plsc

_N = 2048
_H = 8
_N_SPATIAL = 32
_TAB = (2 * 16 + 1) * _N_SPATIAL
_L = 16
_NW = 32
_ROWS_PER_W = _N // _NW
_VECS = _N // _L


def _spatial_d2_thresholds():
    log_c = np.log(np.float32(257.0)).astype(np.float32)
    bins = np.exp(np.linspace(np.float32(0.0), log_c, _N_SPATIAL,
                              dtype=np.float32)).astype(np.float32)

    def mk(b):
        x = np.float32(np.float64(b) ** 2)
        for _ in range(8):
            x = np.nextafter(x, np.float32(-1), dtype=np.float32)
        while not (np.float32(np.sqrt(x)) > b):
            x = np.nextafter(x, np.float32(np.inf), dtype=np.float32)
        return x

    return [float(mk(b)) for b in bins[:_N_SPATIAL - 1]]


_M_THRESH = _spatial_d2_thresholds()

_mesh = plsc.VectorSubcoreMesh(core_axis_name="c", subcore_axis_name="s")


@functools.partial(
    pl.kernel,
    out_type=jax.ShapeDtypeStruct((_H, _N, _N), jnp.float32),
    mesh=_mesh,
    compiler_params=pltpu.CompilerParams(use_tc_tiling_on_sc=False,
                                          needs_layout_passes=False),
    scratch_types=[
        pltpu.VMEM((_H * _TAB,), jnp.float32),
        pltpu.VMEM((_N + _L,), jnp.float32),
        pltpu.VMEM((_N + _L,), jnp.float32),
        pltpu.VMEM((_N + _L,), jnp.float32),
        pltpu.VMEM((_H, _N), jnp.float32),
        pltpu.SemaphoreType.DMA,
    ],
)
def _sc_mask_kernel(coords_t, bias_t, out, tab, ff, px, py, ob, sem):
    wid = lax.axis_index("s") * 2 + lax.axis_index("c")
    base = wid * _ROWS_PER_W

    pltpu.sync_copy(bias_t, tab)
    pltpu.sync_copy(coords_t.at[0], ff.at[pl.ds(0, _N)])
    pltpu.sync_copy(coords_t.at[1], px.at[pl.ds(0, _N)])
    pltpu.sync_copy(coords_t.at[2], py.at[pl.ds(0, _N)])

    def row_body(r, carry):
        i = base + r
        fi = jnp.full((_L,), ff[pl.ds(i, _L)][0], jnp.float32)
        xi = jnp.full((_L,), px[pl.ds(i, _L)][0], jnp.float32)
        yi = jnp.full((_L,), py[pl.ds(i, _L)][0], jnp.float32)

        def vec_body(v, c):
            o = v * _L
            xj = px[pl.ds(o, _L)]
            yj = py[pl.ds(o, _L)]
            fj = ff[pl.ds(o, _L)]
            dx = xj - xi
            dy = yj - yi
            d2 = dx * dx + dy * dy
            s = jnp.zeros((_L,), jnp.int32)
            for m in _M_THRESH:
                s = s + jnp.where(d2 >= m, 1, 0)
            t_diff = fj - fi
            yhalf = jnp.clip(t_diff * 0.5, -17.0, 17.0)
            tr = yhalf.astype(jnp.int32)
            ceil = tr + jnp.where(tr.astype(jnp.float32) * 2.0 < t_diff, 1, 0)
            tbin = jnp.clip(ceil + 16, 0, 32)
            idx = s + tbin * _N_SPATIAL
            for h in range(_H):
                val = plsc.load_gather(tab, [idx + (h * _TAB)])
                ob[h, pl.ds(o, _L)] = val
            return c

        lax.fori_loop(0, _VECS, vec_body, 0)
        copies = [pltpu.async_copy(ob.at[h], out.at[h, i], sem)
                  for h in range(_H)]
        for cp in copies:
            cp.wait()
        return carry

    lax.fori_loop(0, _ROWS_PER_W, row_body, 0)


def kernel(coords, bias):
    coords_t = coords.T
    bias_t = bias.T.reshape(-1)
    return _sc_mask_kernel(coords_t, bias_t)
